# baseline (speedup 1.0000x reference)
import functools

import jax
import jax.numpy as jnp
from jax import lax
from jax.experimental import pallas as pl
from jax.experimental.pallas import tpu as pltpu

NEG_SLOPE = 0.2
EPS = 1e-6
K = 20


def _vn_leaky_from_pd(p, d, axis):
    dot = jnp.sum(p * d, axis=axis, keepdims=True)
    d_sq = jnp.sum(d * d, axis=axis, keepdims=True)
    coef = jnp.where(dot < 0, dot / (d_sq + EPS), 0.0)
    return p - (1.0 - NEG_SLOPE) * coef * d


def _final_body(wf_ref, wd_ref, xc_ref, o_ref):
    xb = xc_ref[0]
    c, three, nb = xb.shape
    x2 = xb.reshape(c, three * nb)
    p = jnp.dot(wf_ref[...], x2, preferred_element_type=jnp.float32)
    d = jnp.dot(wd_ref[...], x2, preferred_element_type=jnp.float32)
    p = p.reshape(-1, three, nb)
    d = d.reshape(-1, three, nb)
    o_ref[0] = _vn_leaky_from_pd(p, d, axis=1)


def _final_layer(xc, Wlf, Wld):
    B, C, _, N = xc.shape
    O = Wlf.shape[0]
    NB = 512
    grid = (B, N // NB)
    return pl.pallas_call(
        _final_body,
        grid=grid,
        in_specs=[
            pl.BlockSpec((O, C), lambda b, n: (0, 0)),
            pl.BlockSpec((O, C), lambda b, n: (0, 0)),
            pl.BlockSpec((1, C, 3, NB), lambda b, n: (b, 0, 0, n)),
        ],
        out_specs=pl.BlockSpec((1, O, 3, NB), lambda b, n: (b, 0, 0, n)),
        out_shape=jax.ShapeDtypeStruct((B, O, 3, N), jnp.float32),
    )(Wlf, Wld, xc)


def _graph_layer(x, Wf, Wd):
    B, C, _, N = x.shape
    O = Wf.shape[0]
    xf = x.reshape(B, C * 3, N)
    inner = -2.0 * jnp.einsum('bcn,bcm->bnm', xf, xf)
    xx = jnp.sum(xf ** 2, axis=1, keepdims=True)
    pd = -xx - inner - jnp.swapaxes(xx, 2, 1)
    idx = lax.top_k(pd, K)[1]

    Wfa, Wfb = Wf[:, :C], Wf[:, C:]
    Wda, Wdb = Wd[:, :C], Wd[:, C:]
    hp = jax.lax.Precision.HIGHEST
    U = jnp.einsum('oc,bcdn->bnod', Wfa, x, precision=hp)
    V = jnp.einsum('oc,bcdn->bnod', Wfb - Wfa, x, precision=hp)
    S = jnp.einsum('oc,bcdn->bnod', Wda, x, precision=hp)
    T = jnp.einsum('oc,bcdn->bnod', Wdb - Wda, x, precision=hp)

    Ug = jnp.take_along_axis(U[:, :, None], idx[..., None, None], axis=1)
    Sg = jnp.take_along_axis(S[:, :, None], idx[..., None, None], axis=1)
    p = Ug + V[:, :, None]
    d = Sg + T[:, :, None]
    out = _vn_leaky_from_pd(p, d, axis=4)
    h = jnp.mean(out, axis=2)
    return jnp.transpose(h, (0, 2, 3, 1))


def _ref_vn_linear_leaky(Wf, Wd, x):
    p = jnp.einsum('oc,bc...->bo...', Wf, x)
    d = jnp.einsum('oc,bc...->bo...', Wd, x)
    dot = jnp.sum(p * d, axis=2, keepdims=True)
    mask = (dot >= 0).astype(x.dtype)
    d_sq = jnp.sum(d * d, axis=2, keepdims=True)
    return NEG_SLOPE * p + (1.0 - NEG_SLOPE) * (
        mask * p + (1.0 - mask) * (p - (dot / (d_sq + EPS)) * d))


def _ref_graph_feature(x, k):
    B = x.shape[0]
    N = x.shape[3]
    xf = x.reshape(B, -1, N)
    inner = -2.0 * jnp.einsum('bcn,bcm->bnm', xf, xf)
    xx = jnp.sum(xf ** 2, axis=1, keepdims=True)
    pd = -xx - inner - jnp.swapaxes(xx, 2, 1)
    idx = lax.top_k(pd, k)[1]
    idx = idx + jnp.arange(B, dtype=idx.dtype)[:, None, None] * N
    idx = idx.reshape(-1)
    num_dims = xf.shape[1] // 3
    xt = jnp.swapaxes(xf, 2, 1).reshape(B * N, -1)
    feature = jnp.take(xt, idx, axis=0).reshape(B, N, k, num_dims, 3)
    xr = xt.reshape(B, N, 1, num_dims, 3)
    xr = jnp.broadcast_to(xr, (B, N, k, num_dims, 3))
    feature = jnp.concatenate([feature - xr, xr], axis=3)
    return jnp.transpose(feature, (0, 3, 4, 1, 2))


def _ref_layer(x, Wf, Wd):
    f = _ref_graph_feature(x, K)
    return jnp.mean(_ref_vn_linear_leaky(Wf, Wd, f), axis=-1)


def kernel(x, W1f, W1d, W2f, W2d, Wlf, Wld):
    h1 = _ref_layer(x, W1f, W1d)
    h2 = _ref_layer(h1, W2f, W2d)
    xc = jnp.concatenate([x, h1, h2], axis=1)
    return _final_layer(xc, Wlf, Wld)


# speedup vs baseline: 1.1720x; 1.1720x over previous
import functools

import jax
import jax.numpy as jnp
from jax import lax
from jax.experimental import pallas as pl
from jax.experimental.pallas import tpu as pltpu
from jax.experimental.pallas import tpu_sc as plsc

NEG_SLOPE = 0.2
EPS = 1e-6
K = 20


def _vn_leaky_from_pd(p, d, axis):
    dot = jnp.sum(p * d, axis=axis, keepdims=True)
    d_sq = jnp.sum(d * d, axis=axis, keepdims=True)
    coef = jnp.where(dot < 0, dot / (d_sq + EPS), 0.0)
    return p - (1.0 - NEG_SLOPE) * coef * d


def _final_body(wf_ref, wd_ref, xc_ref, o_ref):
    xb = xc_ref[0]
    c, three, nb = xb.shape
    x2 = xb.reshape(c, three * nb)
    p = jnp.dot(wf_ref[...], x2, preferred_element_type=jnp.float32)
    d = jnp.dot(wd_ref[...], x2, preferred_element_type=jnp.float32)
    p = p.reshape(-1, three, nb)
    d = d.reshape(-1, three, nb)
    o_ref[0] = _vn_leaky_from_pd(p, d, axis=1)


def _final_layer(xc, Wlf, Wld):
    B, C, _, N = xc.shape
    O = Wlf.shape[0]
    NB = 512
    grid = (B, N // NB)
    return pl.pallas_call(
        _final_body,
        grid=grid,
        in_specs=[
            pl.BlockSpec((O, C), lambda b, n: (0, 0)),
            pl.BlockSpec((O, C), lambda b, n: (0, 0)),
            pl.BlockSpec((1, C, 3, NB), lambda b, n: (b, 0, 0, n)),
        ],
        out_specs=pl.BlockSpec((1, O, 3, NB), lambda b, n: (b, 0, 0, n)),
        out_shape=jax.ShapeDtypeStruct((B, O, 3, N), jnp.float32),
    )(Wlf, Wld, xc)


def _ref_vn_linear_leaky(Wf, Wd, x):
    p = jnp.einsum('oc,bc...->bo...', Wf, x)
    d = jnp.einsum('oc,bc...->bo...', Wd, x)
    dot = jnp.sum(p * d, axis=2, keepdims=True)
    mask = (dot >= 0).astype(x.dtype)
    d_sq = jnp.sum(d * d, axis=2, keepdims=True)
    return NEG_SLOPE * p + (1.0 - NEG_SLOPE) * (
        mask * p + (1.0 - mask) * (p - (dot / (d_sq + EPS)) * d))


def _knn_idx(x):
    B = x.shape[0]
    N = x.shape[3]
    xf = x.reshape(B, -1, N)
    inner = -2.0 * jnp.einsum('bcn,bcm->bnm', xf, xf)
    xx = jnp.sum(xf ** 2, axis=1, keepdims=True)
    pd = -xx - inner - jnp.swapaxes(xx, 2, 1)
    return lax.top_k(pd, K)[1]


def _ref_layer(x, Wf, Wd):
    B = x.shape[0]
    N = x.shape[3]
    idx = _knn_idx(x)
    xf = x.reshape(B, -1, N)
    gidx = (idx + jnp.arange(B, dtype=idx.dtype)[:, None, None] * N).reshape(-1)
    num_dims = xf.shape[1] // 3
    xt = jnp.swapaxes(xf, 2, 1).reshape(B * N, -1)
    feature = jnp.take(xt, gidx, axis=0).reshape(B, N, K, num_dims, 3)
    xr = xt.reshape(B, N, 1, num_dims, 3)
    xr = jnp.broadcast_to(xr, (B, N, K, num_dims, 3))
    feature = jnp.concatenate([feature - xr, xr], axis=3)
    feature = jnp.transpose(feature, (0, 3, 4, 1, 2))
    return jnp.mean(_ref_vn_linear_leaky(Wf, Wd, feature), axis=-1)


_P = 8
_NC = 2
_NS = 16


def _sc_edge_body(table_hbm, vt_hbm, idx_hbm, out_hbm,
                  idx_v, rows_v, vt_v, out_v, gsem):
    nc = _NC
    wid = lax.axis_index("s") * nc + lax.axis_index("c")
    npts = out_hbm.shape[0]
    per_w = npts // (nc * _NS)
    base = wid * per_w
    nchunks = per_w // _P
    inv_k = 1.0 / K
    slope = 1.0 - NEG_SLOPE

    def chunk_body(g, carry):
        pts = base + g * _P
        pltpu.sync_copy(idx_hbm.at[pl.ds(pts * K, _P * K)], idx_v)
        pltpu.async_copy(table_hbm.at[idx_v], rows_v, gsem).wait()
        pltpu.sync_copy(vt_hbm.at[pl.ds(pts, _P)], vt_v)

        def point_body(p, carry2):
            v = [vt_v[p, pl.ds(i * 16, 16)] for i in range(6)]
            t = [vt_v[p, pl.ds(96 + i * 16, 16)] for i in range(6)]

            def edge_body(k, acc):
                accU, accC = acc
                e = p * K + k
                u = [rows_v[e, pl.ds(i * 16, 16)] for i in range(6)]
                s = [rows_v[e, pl.ds(96 + i * 16, 16)] for i in range(6)]
                pp = [u[i] + v[i] for i in range(6)]
                dd = [s[i] + t[i] for i in range(6)]
                dot = [pp[0] * dd[0] + pp[2] * dd[2] + pp[4] * dd[4],
                       pp[1] * dd[1] + pp[3] * dd[3] + pp[5] * dd[5]]
                dsq = [dd[0] * dd[0] + dd[2] * dd[2] + dd[4] * dd[4],
                       dd[1] * dd[1] + dd[3] * dd[3] + dd[5] * dd[5]]
                coef = [jnp.where(dot[i] < 0.0, dot[i] / (dsq[i] + EPS), 0.0)
                        for i in range(2)]
                accU = [accU[i] + u[i] for i in range(6)]
                accC = [accC[i] + coef[i % 2] * dd[i] for i in range(6)]
                return (accU, accC)

            zeros = [jnp.zeros((16,), jnp.float32) for _ in range(6)]
            accU, accC = lax.fori_loop(0, K, edge_body, (zeros, list(zeros)))
            for i in range(6):
                h = v[i] + inv_k * accU[i] - (slope * inv_k) * accC[i]
                out_v[p, pl.ds(i * 16, 16)] = h
            return carry2

        lax.fori_loop(0, _P, point_body, 0)
        pltpu.sync_copy(out_v, out_hbm.at[pl.ds(pts, _P)])
        return carry

    lax.fori_loop(0, nchunks, chunk_body, 0)


def _sc_edge_layer(table, vt, gidx):
    npts = table.shape[0]
    mesh = plsc.VectorSubcoreMesh(core_axis_name="c", subcore_axis_name="s",
                                  num_cores=_NC, num_subcores=_NS)
    f = functools.partial(
        pl.kernel, mesh=mesh,
        compiler_params=pltpu.CompilerParams(use_tc_tiling_on_sc=False),
        out_type=jax.ShapeDtypeStruct((npts, 96), jnp.float32),
        scratch_types=[
            pltpu.VMEM((_P * K,), jnp.int32),
            pltpu.VMEM((_P * K, 192), jnp.float32),
            pltpu.VMEM((_P, 192), jnp.float32),
            pltpu.VMEM((_P, 96), jnp.float32),
            pltpu.SemaphoreType.DMA,
        ],
    )(_sc_edge_body)
    return f(table, vt, gidx)


def _graph_layer2(h1, Wf, Wd):
    B, C, _, N = h1.shape
    O = Wf.shape[0]
    idx = _knn_idx(h1)
    gidx = (idx + jnp.arange(B, dtype=idx.dtype)[:, None, None] * N)
    gidx = gidx.reshape(-1).astype(jnp.int32)

    hp = lax.Precision.HIGHEST
    Wfa, Wfb = Wf[:, :C], Wf[:, C:]
    Wda, Wdb = Wd[:, :C], Wd[:, C:]
    U = jnp.einsum('oc,bcdn->bndo', Wfa, h1, precision=hp)
    V = jnp.einsum('oc,bcdn->bndo', Wfb - Wfa, h1, precision=hp)
    S = jnp.einsum('oc,bcdn->bndo', Wda, h1, precision=hp)
    T = jnp.einsum('oc,bcdn->bndo', Wdb - Wda, h1, precision=hp)

    def pack(a, b):
        ab = jnp.stack([a, b], axis=2)
        ab = jnp.pad(ab, ((0, 0), (0, 0), (0, 0), (0, 0), (0, 32 - O)))
        return ab.reshape(B * N, 192)

    table = pack(U, S)
    vt = pack(V, T)
    h = _sc_edge_layer(table, vt, gidx)
    h = h.reshape(B, N, 3, 32)[:, :, :, :O]
    return jnp.transpose(h, (0, 3, 2, 1))


def kernel(x, W1f, W1d, W2f, W2d, Wlf, Wld):
    h1 = _ref_layer(x, W1f, W1d)
    h2 = _graph_layer2(h1, W2f, W2d)
    xc = jnp.concatenate([x, h1, h2], axis=1)
    return _final_layer(xc, Wlf, Wld)


# speedup vs baseline: 2.7888x; 2.3796x over previous
import functools

import jax
import jax.numpy as jnp
from jax import lax
from jax.experimental import pallas as pl
from jax.experimental.pallas import tpu as pltpu
from jax.experimental.pallas import tpu_sc as plsc

NEG_SLOPE = 0.2
EPS = 1e-6
K = 20


def _vn_leaky_from_pd(p, d, axis):
    dot = jnp.sum(p * d, axis=axis, keepdims=True)
    d_sq = jnp.sum(d * d, axis=axis, keepdims=True)
    coef = jnp.where(dot < 0, dot / (d_sq + EPS), 0.0)
    return p - (1.0 - NEG_SLOPE) * coef * d


def _final_body(wf_ref, wd_ref, xc_ref, o_ref):
    xb = xc_ref[0]
    c, three, nb = xb.shape
    x2 = xb.reshape(c, three * nb)
    p = jnp.dot(wf_ref[...], x2, preferred_element_type=jnp.float32)
    d = jnp.dot(wd_ref[...], x2, preferred_element_type=jnp.float32)
    p = p.reshape(-1, three, nb)
    d = d.reshape(-1, three, nb)
    o_ref[0] = _vn_leaky_from_pd(p, d, axis=1)


def _final_layer(xc, Wlf, Wld):
    B, C, _, N = xc.shape
    O = Wlf.shape[0]
    NB = 512
    grid = (B, N // NB)
    return pl.pallas_call(
        _final_body,
        grid=grid,
        in_specs=[
            pl.BlockSpec((O, C), lambda b, n: (0, 0)),
            pl.BlockSpec((O, C), lambda b, n: (0, 0)),
            pl.BlockSpec((1, C, 3, NB), lambda b, n: (b, 0, 0, n)),
        ],
        out_specs=pl.BlockSpec((1, O, 3, NB), lambda b, n: (b, 0, 0, n)),
        out_shape=jax.ShapeDtypeStruct((B, O, 3, N), jnp.float32),
    )(Wlf, Wld, xc)


def _ref_vn_linear_leaky(Wf, Wd, x):
    p = jnp.einsum('oc,bc...->bo...', Wf, x)
    d = jnp.einsum('oc,bc...->bo...', Wd, x)
    dot = jnp.sum(p * d, axis=2, keepdims=True)
    mask = (dot >= 0).astype(x.dtype)
    d_sq = jnp.sum(d * d, axis=2, keepdims=True)
    return NEG_SLOPE * p + (1.0 - NEG_SLOPE) * (
        mask * p + (1.0 - mask) * (p - (dot / (d_sq + EPS)) * d))


def _knn_idx(x):
    B = x.shape[0]
    N = x.shape[3]
    xf = x.reshape(B, -1, N)
    inner = -2.0 * jnp.einsum('bcn,bcm->bnm', xf, xf)
    xx = jnp.sum(xf ** 2, axis=1, keepdims=True)
    pd = -xx - inner - jnp.swapaxes(xx, 2, 1)
    nch = N // 16
    pdc = pd.reshape(B, N, nch, 16)
    cm = jnp.max(pdc, axis=-1)
    cidx = lax.top_k(cm, K)[1]
    g = jnp.take_along_axis(pdc, cidx[..., None], axis=2)
    gi = cidx[..., None] * 16 + jnp.arange(16, dtype=cidx.dtype)
    i2 = lax.top_k(g.reshape(B, N, K * 16), K)[1]
    return jnp.take_along_axis(gi.reshape(B, N, K * 16), i2, axis=-1)


def _ref_layer(x, Wf, Wd):
    B = x.shape[0]
    N = x.shape[3]
    idx = _knn_idx(x)
    xf = x.reshape(B, -1, N)
    gidx = (idx + jnp.arange(B, dtype=idx.dtype)[:, None, None] * N).reshape(-1)
    num_dims = xf.shape[1] // 3
    xt = jnp.swapaxes(xf, 2, 1).reshape(B * N, -1)
    feature = jnp.take(xt, gidx, axis=0).reshape(B, N, K, num_dims, 3)
    xr = xt.reshape(B, N, 1, num_dims, 3)
    xr = jnp.broadcast_to(xr, (B, N, K, num_dims, 3))
    feature = jnp.concatenate([feature - xr, xr], axis=3)
    feature = jnp.transpose(feature, (0, 3, 4, 1, 2))
    return jnp.mean(_ref_vn_linear_leaky(Wf, Wd, feature), axis=-1)


_P = 8
_NC = 2
_NS = 16


def _sc_edge_body(table_hbm, vt_hbm, idx_hbm, out_hbm,
                  idx_v, rows_v, vt_v, out_v, gsem):
    nc = _NC
    wid = lax.axis_index("s") * nc + lax.axis_index("c")
    npts = out_hbm.shape[0]
    per_w = npts // (nc * _NS)
    base = wid * per_w
    nchunks = per_w // _P
    inv_k = 1.0 / K
    slope = 1.0 - NEG_SLOPE

    def chunk_body(g, carry):
        pts = base + g * _P
        pltpu.sync_copy(idx_hbm.at[pl.ds(pts * K, _P * K)], idx_v)
        pltpu.async_copy(table_hbm.at[idx_v], rows_v, gsem).wait()
        pltpu.sync_copy(vt_hbm.at[pl.ds(pts, _P)], vt_v)

        def point_body(p, carry2):
            v = [vt_v[p, pl.ds(i * 16, 16)] for i in range(6)]
            t = [vt_v[p, pl.ds(96 + i * 16, 16)] for i in range(6)]

            def edge_body(k, acc):
                accU, accC = acc
                e = p * K + k
                u = [rows_v[e, pl.ds(i * 16, 16)] for i in range(6)]
                s = [rows_v[e, pl.ds(96 + i * 16, 16)] for i in range(6)]
                pp = [u[i] + v[i] for i in range(6)]
                dd = [s[i] + t[i] for i in range(6)]
                dot = [pp[0] * dd[0] + pp[2] * dd[2] + pp[4] * dd[4],
                       pp[1] * dd[1] + pp[3] * dd[3] + pp[5] * dd[5]]
                dsq = [dd[0] * dd[0] + dd[2] * dd[2] + dd[4] * dd[4],
                       dd[1] * dd[1] + dd[3] * dd[3] + dd[5] * dd[5]]
                coef = [jnp.where(dot[i] < 0.0, dot[i] / (dsq[i] + EPS), 0.0)
                        for i in range(2)]
                accU = [accU[i] + u[i] for i in range(6)]
                accC = [accC[i] + coef[i % 2] * dd[i] for i in range(6)]
                return (accU, accC)

            zeros = [jnp.zeros((16,), jnp.float32) for _ in range(6)]
            accU, accC = lax.fori_loop(0, K, edge_body, (zeros, list(zeros)))
            for i in range(6):
                h = v[i] + inv_k * accU[i] - (slope * inv_k) * accC[i]
                out_v[p, pl.ds(i * 16, 16)] = h
            return carry2

        lax.fori_loop(0, _P, point_body, 0)
        pltpu.sync_copy(out_v, out_hbm.at[pl.ds(pts, _P)])
        return carry

    lax.fori_loop(0, nchunks, chunk_body, 0)


def _sc_edge_layer(table, vt, gidx):
    npts = table.shape[0]
    mesh = plsc.VectorSubcoreMesh(core_axis_name="c", subcore_axis_name="s",
                                  num_cores=_NC, num_subcores=_NS)
    f = functools.partial(
        pl.kernel, mesh=mesh,
        compiler_params=pltpu.CompilerParams(use_tc_tiling_on_sc=False),
        out_type=jax.ShapeDtypeStruct((npts, 96), jnp.float32),
        scratch_types=[
            pltpu.VMEM((_P * K,), jnp.int32),
            pltpu.VMEM((_P * K, 192), jnp.float32),
            pltpu.VMEM((_P, 192), jnp.float32),
            pltpu.VMEM((_P, 96), jnp.float32),
            pltpu.SemaphoreType.DMA,
        ],
    )(_sc_edge_body)
    return f(table, vt, gidx)


def _graph_layer2(h1, Wf, Wd):
    B, C, _, N = h1.shape
    O = Wf.shape[0]
    idx = _knn_idx(h1)
    gidx = (idx + jnp.arange(B, dtype=idx.dtype)[:, None, None] * N)
    gidx = gidx.reshape(-1).astype(jnp.int32)

    hp = lax.Precision.HIGHEST
    Wfa, Wfb = Wf[:, :C], Wf[:, C:]
    Wda, Wdb = Wd[:, :C], Wd[:, C:]
    U = jnp.einsum('oc,bcdn->bndo', Wfa, h1, precision=hp)
    V = jnp.einsum('oc,bcdn->bndo', Wfb - Wfa, h1, precision=hp)
    S = jnp.einsum('oc,bcdn->bndo', Wda, h1, precision=hp)
    T = jnp.einsum('oc,bcdn->bndo', Wdb - Wda, h1, precision=hp)

    def pack(a, b):
        ab = jnp.stack([a, b], axis=2)
        ab = jnp.pad(ab, ((0, 0), (0, 0), (0, 0), (0, 0), (0, 32 - O)))
        return ab.reshape(B * N, 192)

    table = pack(U, S)
    vt = pack(V, T)
    h = _sc_edge_layer(table, vt, gidx)
    h = h.reshape(B, N, 3, 32)[:, :, :, :O]
    return jnp.transpose(h, (0, 3, 2, 1))


def kernel(x, W1f, W1d, W2f, W2d, Wlf, Wld):
    h1 = _ref_layer(x, W1f, W1d)
    h2 = _graph_layer2(h1, W2f, W2d)
    xc = jnp.concatenate([x, h1, h2], axis=1)
    return _final_layer(xc, Wlf, Wld)


# speedup vs baseline: 2.8460x; 1.0205x over previous
import functools

import jax
import jax.numpy as jnp
from jax import lax
from jax.experimental import pallas as pl
from jax.experimental.pallas import tpu as pltpu
from jax.experimental.pallas import tpu_sc as plsc

NEG_SLOPE = 0.2
EPS = 1e-6
K = 20


def _vn_leaky_from_pd(p, d, axis):
    dot = jnp.sum(p * d, axis=axis, keepdims=True)
    d_sq = jnp.sum(d * d, axis=axis, keepdims=True)
    coef = jnp.where(dot < 0, dot / (d_sq + EPS), 0.0)
    return p - (1.0 - NEG_SLOPE) * coef * d


def _final_body(wf_ref, wd_ref, xc_ref, o_ref):
    xb = xc_ref[0]
    c, three, nb = xb.shape
    x2 = xb.reshape(c, three * nb)
    p = jnp.dot(wf_ref[...], x2, preferred_element_type=jnp.float32)
    d = jnp.dot(wd_ref[...], x2, preferred_element_type=jnp.float32)
    p = p.reshape(-1, three, nb)
    d = d.reshape(-1, three, nb)
    o_ref[0] = _vn_leaky_from_pd(p, d, axis=1)


def _final_layer(xc, Wlf, Wld):
    B, C, _, N = xc.shape
    O = Wlf.shape[0]
    NB = 512
    grid = (B, N // NB)
    return pl.pallas_call(
        _final_body,
        grid=grid,
        in_specs=[
            pl.BlockSpec((O, C), lambda b, n: (0, 0)),
            pl.BlockSpec((O, C), lambda b, n: (0, 0)),
            pl.BlockSpec((1, C, 3, NB), lambda b, n: (b, 0, 0, n)),
        ],
        out_specs=pl.BlockSpec((1, O, 3, NB), lambda b, n: (b, 0, 0, n)),
        out_shape=jax.ShapeDtypeStruct((B, O, 3, N), jnp.float32),
    )(Wlf, Wld, xc)


def _ref_vn_linear_leaky(Wf, Wd, x):
    p = jnp.einsum('oc,bc...->bo...', Wf, x)
    d = jnp.einsum('oc,bc...->bo...', Wd, x)
    dot = jnp.sum(p * d, axis=2, keepdims=True)
    mask = (dot >= 0).astype(x.dtype)
    d_sq = jnp.sum(d * d, axis=2, keepdims=True)
    return NEG_SLOPE * p + (1.0 - NEG_SLOPE) * (
        mask * p + (1.0 - mask) * (p - (dot / (d_sq + EPS)) * d))


def _knn_idx(x):
    B = x.shape[0]
    N = x.shape[3]
    xf = x.reshape(B, -1, N)
    inner = -2.0 * jnp.einsum('bcn,bcm->bnm', xf, xf)
    xx = jnp.sum(xf ** 2, axis=1, keepdims=True)
    pd = -xx - inner - jnp.swapaxes(xx, 2, 1)
    nch = N // 16
    pdc = pd.reshape(B, N, nch, 16)
    cm = jnp.max(pdc, axis=-1)
    cidx = lax.top_k(cm, K)[1]
    g = jnp.take_along_axis(pdc, cidx[..., None], axis=2)
    gi = cidx[..., None] * 16 + jnp.arange(16, dtype=cidx.dtype)
    i2 = lax.top_k(g.reshape(B, N, K * 16), K)[1]
    return jnp.take_along_axis(gi.reshape(B, N, K * 16), i2, axis=-1)


def _ref_layer(x, Wf, Wd):
    B = x.shape[0]
    N = x.shape[3]
    idx = _knn_idx(x)
    xf = x.reshape(B, -1, N)
    gidx = (idx + jnp.arange(B, dtype=idx.dtype)[:, None, None] * N).reshape(-1)
    num_dims = xf.shape[1] // 3
    xt = jnp.swapaxes(xf, 2, 1).reshape(B * N, -1)
    feature = jnp.take(xt, gidx, axis=0).reshape(B, N, K, num_dims, 3)
    xr = xt.reshape(B, N, 1, num_dims, 3)
    xr = jnp.broadcast_to(xr, (B, N, K, num_dims, 3))
    feature = jnp.concatenate([feature - xr, xr], axis=3)
    feature = jnp.transpose(feature, (0, 3, 4, 1, 2))
    return jnp.mean(_ref_vn_linear_leaky(Wf, Wd, feature), axis=-1)


_P = 8
_NC = 2
_NS = 16


def _sc_edge_body(table_hbm, vt_hbm, idx_hbm, out_hbm,
                  idx_v0, idx_v1, rows_v0, rows_v1, vt_v, out_v, gsem0, gsem1):
    nc = _NC
    wid = lax.axis_index("s") * nc + lax.axis_index("c")
    npts = out_hbm.shape[0]
    per_w = npts // (nc * _NS)
    base = wid * per_w
    nchunks = per_w // _P
    inv_k = 1.0 / K
    slope = 1.0 - NEG_SLOPE
    idx_b = (idx_v0, idx_v1)
    rows_b = (rows_v0, rows_v1)
    sem_b = (gsem0, gsem1)

    pltpu.sync_copy(idx_hbm.at[pl.ds(base * K, _P * K)], idx_v0)
    pltpu.async_copy(table_hbm.at[idx_v0], rows_v0, gsem0)

    def chunk_pair_body(g2, carry):
        for bsel in range(2):
            _chunk_half(g2, bsel, base, nchunks, inv_k, slope,
                        table_hbm, vt_hbm, idx_hbm, out_hbm,
                        idx_b, rows_b, sem_b, vt_v, out_v)
        return carry

    lax.fori_loop(0, nchunks // 2, chunk_pair_body, 0)


def _chunk_half(g2, bsel, base, nchunks, inv_k, slope,
                table_hbm, vt_hbm, idx_hbm, out_hbm,
                idx_b, rows_b, sem_b, vt_v, out_v):
    g = 2 * g2 + bsel
    pts = base + g * _P
    rows_v = rows_b[bsel]
    pltpu.make_async_copy(table_hbm.at[idx_b[bsel]], rows_v,
                          sem_b[bsel]).wait()
    nxt = g + 1

    @pl.when(nxt < nchunks)
    def _():
        pts_n = base + nxt * _P
        o = 1 - bsel
        pltpu.sync_copy(idx_hbm.at[pl.ds(pts_n * K, _P * K)], idx_b[o])
        pltpu.async_copy(table_hbm.at[idx_b[o]], rows_b[o], sem_b[o])

    pltpu.sync_copy(vt_hbm.at[pl.ds(pts, _P)], vt_v)

    if True:
        def point_body(p, carry2):
            v = [vt_v[p, pl.ds(i * 16, 16)] for i in range(6)]
            t = [vt_v[p, pl.ds(96 + i * 16, 16)] for i in range(6)]

            def edge_body(k, acc):
                accU, accC = acc
                e = p * K + k
                u = [rows_v[e, pl.ds(i * 16, 16)] for i in range(6)]
                s = [rows_v[e, pl.ds(96 + i * 16, 16)] for i in range(6)]
                pp = [u[i] + v[i] for i in range(6)]
                dd = [s[i] + t[i] for i in range(6)]
                dot = [pp[0] * dd[0] + pp[2] * dd[2] + pp[4] * dd[4],
                       pp[1] * dd[1] + pp[3] * dd[3] + pp[5] * dd[5]]
                dsq = [dd[0] * dd[0] + dd[2] * dd[2] + dd[4] * dd[4],
                       dd[1] * dd[1] + dd[3] * dd[3] + dd[5] * dd[5]]
                coef = [jnp.where(dot[i] < 0.0, dot[i] / (dsq[i] + EPS), 0.0)
                        for i in range(2)]
                accU = [accU[i] + u[i] for i in range(6)]
                accC = [accC[i] + coef[i % 2] * dd[i] for i in range(6)]
                return (accU, accC)

            zeros = [jnp.zeros((16,), jnp.float32) for _ in range(6)]
            accU, accC = lax.fori_loop(0, K, edge_body, (zeros, list(zeros)))
            for i in range(6):
                h = v[i] + inv_k * accU[i] - (slope * inv_k) * accC[i]
                out_v[p, pl.ds(i * 16, 16)] = h
            return carry2

        lax.fori_loop(0, _P, point_body, 0)
        pltpu.sync_copy(out_v, out_hbm.at[pl.ds(pts, _P)])


def _sc_edge_layer(table, vt, gidx):
    npts = table.shape[0]
    mesh = plsc.VectorSubcoreMesh(core_axis_name="c", subcore_axis_name="s",
                                  num_cores=_NC, num_subcores=_NS)
    f = functools.partial(
        pl.kernel, mesh=mesh,
        compiler_params=pltpu.CompilerParams(use_tc_tiling_on_sc=False),
        out_type=jax.ShapeDtypeStruct((npts, 96), jnp.float32),
        scratch_types=[
            pltpu.VMEM((_P * K,), jnp.int32),
            pltpu.VMEM((_P * K,), jnp.int32),
            pltpu.VMEM((_P * K, 192), jnp.float32),
            pltpu.VMEM((_P * K, 192), jnp.float32),
            pltpu.VMEM((_P, 192), jnp.float32),
            pltpu.VMEM((_P, 96), jnp.float32),
            pltpu.SemaphoreType.DMA,
            pltpu.SemaphoreType.DMA,
        ],
    )(_sc_edge_body)
    return f(table, vt, gidx)


def _graph_layer2(h1, Wf, Wd):
    B, C, _, N = h1.shape
    O = Wf.shape[0]
    idx = _knn_idx(h1)
    gidx = (idx + jnp.arange(B, dtype=idx.dtype)[:, None, None] * N)
    gidx = gidx.reshape(-1).astype(jnp.int32)

    hp = lax.Precision.HIGHEST
    Wfa, Wfb = Wf[:, :C], Wf[:, C:]
    Wda, Wdb = Wd[:, :C], Wd[:, C:]
    U = jnp.einsum('oc,bcdn->bndo', Wfa, h1, precision=hp)
    V = jnp.einsum('oc,bcdn->bndo', Wfb - Wfa, h1, precision=hp)
    S = jnp.einsum('oc,bcdn->bndo', Wda, h1, precision=hp)
    T = jnp.einsum('oc,bcdn->bndo', Wdb - Wda, h1, precision=hp)

    def pack(a, b):
        ab = jnp.stack([a, b], axis=2)
        ab = jnp.pad(ab, ((0, 0), (0, 0), (0, 0), (0, 0), (0, 32 - O)))
        return ab.reshape(B * N, 192)

    table = pack(U, S)
    vt = pack(V, T)
    h = _sc_edge_layer(table, vt, gidx)
    h = h.reshape(B, N, 3, 32)[:, :, :, :O]
    return jnp.transpose(h, (0, 3, 2, 1))


def kernel(x, W1f, W1d, W2f, W2d, Wlf, Wld):
    h1 = _ref_layer(x, W1f, W1d)
    h2 = _graph_layer2(h1, W2f, W2d)
    xc = jnp.concatenate([x, h1, h2], axis=1)
    return _final_layer(xc, Wlf, Wld)


# speedup vs baseline: 3.0686x; 1.0782x over previous
import functools

import jax
import jax.numpy as jnp
from jax import lax
from jax.experimental import pallas as pl
from jax.experimental.pallas import tpu as pltpu
from jax.experimental.pallas import tpu_sc as plsc

NEG_SLOPE = 0.2
EPS = 1e-6
K = 20


def _vn_leaky_from_pd(p, d, axis):
    dot = jnp.sum(p * d, axis=axis, keepdims=True)
    d_sq = jnp.sum(d * d, axis=axis, keepdims=True)
    coef = jnp.where(dot < 0, dot / (d_sq + EPS), 0.0)
    return p - (1.0 - NEG_SLOPE) * coef * d


def _final_body(wf_ref, wd_ref, xc_ref, o_ref):
    xb = xc_ref[0]
    c, three, nb = xb.shape
    x2 = xb.reshape(c, three * nb)
    p = jnp.dot(wf_ref[...], x2, preferred_element_type=jnp.float32)
    d = jnp.dot(wd_ref[...], x2, preferred_element_type=jnp.float32)
    p = p.reshape(-1, three, nb)
    d = d.reshape(-1, three, nb)
    o_ref[0] = _vn_leaky_from_pd(p, d, axis=1)


def _final_layer(xc, Wlf, Wld):
    B, C, _, N = xc.shape
    O = Wlf.shape[0]
    NB = 512
    grid = (B, N // NB)
    return pl.pallas_call(
        _final_body,
        grid=grid,
        in_specs=[
            pl.BlockSpec((O, C), lambda b, n: (0, 0)),
            pl.BlockSpec((O, C), lambda b, n: (0, 0)),
            pl.BlockSpec((1, C, 3, NB), lambda b, n: (b, 0, 0, n)),
        ],
        out_specs=pl.BlockSpec((1, O, 3, NB), lambda b, n: (b, 0, 0, n)),
        out_shape=jax.ShapeDtypeStruct((B, O, 3, N), jnp.float32),
    )(Wlf, Wld, xc)


def _ref_vn_linear_leaky(Wf, Wd, x):
    p = jnp.einsum('oc,bc...->bo...', Wf, x)
    d = jnp.einsum('oc,bc...->bo...', Wd, x)
    dot = jnp.sum(p * d, axis=2, keepdims=True)
    mask = (dot >= 0).astype(x.dtype)
    d_sq = jnp.sum(d * d, axis=2, keepdims=True)
    return NEG_SLOPE * p + (1.0 - NEG_SLOPE) * (
        mask * p + (1.0 - mask) * (p - (dot / (d_sq + EPS)) * d))


def _knn_idx(x):
    B = x.shape[0]
    N = x.shape[3]
    xf = x.reshape(B, -1, N)
    inner = -2.0 * jnp.einsum('bcn,bcm->bnm', xf, xf)
    xx = jnp.sum(xf ** 2, axis=1, keepdims=True)
    pd = -xx - inner - jnp.swapaxes(xx, 2, 1)
    nch = N // 16
    pdc = pd.reshape(B, N, nch, 16)
    cm = jnp.max(pdc, axis=-1)
    cidx = lax.top_k(cm, K)[1]
    g = jnp.take_along_axis(pdc, cidx[..., None], axis=2, mode="clip")
    gi = cidx[..., None] * 16 + jnp.arange(16, dtype=cidx.dtype)
    i2 = lax.top_k(g.reshape(B, N, K * 16), K)[1]
    return jnp.take_along_axis(gi.reshape(B, N, K * 16), i2, axis=-1,
                               mode="clip")


def _ref_layer(x, Wf, Wd):
    B = x.shape[0]
    N = x.shape[3]
    idx = _knn_idx(x)
    xf = x.reshape(B, -1, N)
    gidx = (idx + jnp.arange(B, dtype=idx.dtype)[:, None, None] * N).reshape(-1)
    num_dims = xf.shape[1] // 3
    xt = jnp.swapaxes(xf, 2, 1).reshape(B * N, -1)
    feature = jnp.take(xt, gidx, axis=0, mode="clip").reshape(B, N, K, num_dims, 3)
    xr = xt.reshape(B, N, 1, num_dims, 3)
    xr = jnp.broadcast_to(xr, (B, N, K, num_dims, 3))
    feature = jnp.concatenate([feature - xr, xr], axis=3)
    feature = jnp.transpose(feature, (0, 3, 4, 1, 2))
    return jnp.mean(_ref_vn_linear_leaky(Wf, Wd, feature), axis=-1)


_P = 8
_NC = 2
_NS = 16


def _sc_edge_body(table_hbm, vt_hbm, idx_hbm, out_hbm,
                  idx_v0, idx_v1, rows_v0, rows_v1, vt_v, out_v, gsem0, gsem1):
    nc = _NC
    wid = lax.axis_index("s") * nc + lax.axis_index("c")
    npts = out_hbm.shape[0]
    per_w = npts // (nc * _NS)
    base = wid * per_w
    nchunks = per_w // _P
    inv_k = 1.0 / K
    slope = 1.0 - NEG_SLOPE
    idx_b = (idx_v0, idx_v1)
    rows_b = (rows_v0, rows_v1)
    sem_b = (gsem0, gsem1)

    pltpu.sync_copy(idx_hbm.at[pl.ds(base * K, _P * K)], idx_v0)
    pltpu.async_copy(table_hbm.at[idx_v0], rows_v0, gsem0)

    def chunk_pair_body(g2, carry):
        for bsel in range(2):
            _chunk_half(g2, bsel, base, nchunks, inv_k, slope,
                        table_hbm, vt_hbm, idx_hbm, out_hbm,
                        idx_b, rows_b, sem_b, vt_v, out_v)
        return carry

    lax.fori_loop(0, nchunks // 2, chunk_pair_body, 0)


def _chunk_half(g2, bsel, base, nchunks, inv_k, slope,
                table_hbm, vt_hbm, idx_hbm, out_hbm,
                idx_b, rows_b, sem_b, vt_v, out_v):
    g = 2 * g2 + bsel
    pts = base + g * _P
    rows_v = rows_b[bsel]
    pltpu.make_async_copy(table_hbm.at[idx_b[bsel]], rows_v,
                          sem_b[bsel]).wait()
    nxt = g + 1

    @pl.when(nxt < nchunks)
    def _():
        pts_n = base + nxt * _P
        o = 1 - bsel
        pltpu.sync_copy(idx_hbm.at[pl.ds(pts_n * K, _P * K)], idx_b[o])
        pltpu.async_copy(table_hbm.at[idx_b[o]], rows_b[o], sem_b[o])

    pltpu.sync_copy(vt_hbm.at[pl.ds(pts, _P)], vt_v)

    if True:
        def point_body(p, carry2):
            v = [vt_v[p, pl.ds(i * 16, 16)] for i in range(6)]
            t = [vt_v[p, pl.ds(96 + i * 16, 16)] for i in range(6)]

            def edge_body(k, acc):
                accU, accC = acc
                e = p * K + k
                u = [rows_v[e, pl.ds(i * 16, 16)] for i in range(6)]
                s = [rows_v[e, pl.ds(96 + i * 16, 16)] for i in range(6)]
                pp = [u[i] + v[i] for i in range(6)]
                dd = [s[i] + t[i] for i in range(6)]
                dot = [pp[0] * dd[0] + pp[2] * dd[2] + pp[4] * dd[4],
                       pp[1] * dd[1] + pp[3] * dd[3] + pp[5] * dd[5]]
                dsq = [dd[0] * dd[0] + dd[2] * dd[2] + dd[4] * dd[4],
                       dd[1] * dd[1] + dd[3] * dd[3] + dd[5] * dd[5]]
                coef = [jnp.where(dot[i] < 0.0, dot[i] / (dsq[i] + EPS), 0.0)
                        for i in range(2)]
                accU = [accU[i] + u[i] for i in range(6)]
                accC = [accC[i] + coef[i % 2] * dd[i] for i in range(6)]
                return (accU, accC)

            zeros = [jnp.zeros((16,), jnp.float32) for _ in range(6)]
            accU, accC = lax.fori_loop(0, K, edge_body, (zeros, list(zeros)))
            for i in range(6):
                h = v[i] + inv_k * accU[i] - (slope * inv_k) * accC[i]
                out_v[p, pl.ds(i * 16, 16)] = h
            return carry2

        lax.fori_loop(0, _P, point_body, 0)
        pltpu.sync_copy(out_v, out_hbm.at[pl.ds(pts, _P)])


def _sc_edge_layer(table, vt, gidx):
    npts = table.shape[0]
    mesh = plsc.VectorSubcoreMesh(core_axis_name="c", subcore_axis_name="s",
                                  num_cores=_NC, num_subcores=_NS)
    f = functools.partial(
        pl.kernel, mesh=mesh,
        compiler_params=pltpu.CompilerParams(use_tc_tiling_on_sc=False),
        out_type=jax.ShapeDtypeStruct((npts, 96), jnp.float32),
        scratch_types=[
            pltpu.VMEM((_P * K,), jnp.int32),
            pltpu.VMEM((_P * K,), jnp.int32),
            pltpu.VMEM((_P * K, 192), jnp.float32),
            pltpu.VMEM((_P * K, 192), jnp.float32),
            pltpu.VMEM((_P, 192), jnp.float32),
            pltpu.VMEM((_P, 96), jnp.float32),
            pltpu.SemaphoreType.DMA,
            pltpu.SemaphoreType.DMA,
        ],
    )(_sc_edge_body)
    return f(table, vt, gidx)


def _graph_layer2(h1, Wf, Wd):
    B, C, _, N = h1.shape
    O = Wf.shape[0]
    idx = _knn_idx(h1)
    gidx = (idx + jnp.arange(B, dtype=idx.dtype)[:, None, None] * N)
    gidx = gidx.reshape(-1).astype(jnp.int32)

    hp = lax.Precision.HIGHEST
    Wfa, Wfb = Wf[:, :C], Wf[:, C:]
    Wda, Wdb = Wd[:, :C], Wd[:, C:]
    U = jnp.einsum('oc,bcdn->bndo', Wfa, h1, precision=hp)
    V = jnp.einsum('oc,bcdn->bndo', Wfb - Wfa, h1, precision=hp)
    S = jnp.einsum('oc,bcdn->bndo', Wda, h1, precision=hp)
    T = jnp.einsum('oc,bcdn->bndo', Wdb - Wda, h1, precision=hp)

    def pack(a, b):
        ab = jnp.stack([a, b], axis=2)
        ab = jnp.pad(ab, ((0, 0), (0, 0), (0, 0), (0, 0), (0, 32 - O)))
        return ab.reshape(B * N, 192)

    table = pack(U, S)
    vt = pack(V, T)
    h = _sc_edge_layer(table, vt, gidx)
    h = h.reshape(B, N, 3, 32)[:, :, :, :O]
    return jnp.transpose(h, (0, 3, 2, 1))


def kernel(x, W1f, W1d, W2f, W2d, Wlf, Wld):
    h1 = _ref_layer(x, W1f, W1d)
    h2 = _graph_layer2(h1, W2f, W2d)
    xc = jnp.concatenate([x, h1, h2], axis=1)
    return _final_layer(xc, Wlf, Wld)
